# baseline (speedup 1.0000x reference)
import functools

import jax
import jax.numpy as jnp
from jax import lax
from jax.experimental import pallas as pl
from jax.experimental.pallas import tpu as pltpu
from jax.experimental.pallas import tpu_sc as plsc

GRID = 64
C_IN = 16
C_OUT = 32
K = 3
OUT = GRID - K + 1
PLANE = GRID * GRID
ROWS = 520
M8 = OUT * GRID // 8
LANES = 8 * C_IN
NOUT = 8 * C_OUT


def _conv_body(d0_ref, d1_ref, d2_ref, w_ref, b_ref, out_ref):
    slabs = (d0_ref, d1_ref, d2_ref)
    shifted = []
    for kx in range(K):
        s = slabs[kx][0]
        row = [s[0:ROWS - 1]]
        for kz in (1, 2):
            sh = jnp.concatenate(
                [s[0:ROWS - 1, 16 * kz:], s[1:ROWS, :16 * kz]], axis=1)
            row.append(sh)
        shifted.append(row)
    acc = jnp.zeros((M8, NOUT), dtype=jnp.float32)
    for kx in range(K):
        for ky in range(K):
            for kz in range(K):
                t = (kx * K + ky) * K + kz
                lhs = shifted[kx][kz][8 * ky:8 * ky + M8, :]
                acc += jnp.dot(lhs, w_ref[t],
                               preferred_element_type=jnp.float32)
    acc += b_ref[0]
    out_ref[0] = jnp.maximum(acc, 0.0).reshape(OUT, 8, NOUT)


@jax.jit
def _conv(dense_p, w_bd, bias_p):
    return pl.pallas_call(
        _conv_body,
        grid=(OUT,),
        in_specs=[
            pl.BlockSpec((1, ROWS, LANES), lambda i: (i, 0, 0)),
            pl.BlockSpec((1, ROWS, LANES), lambda i: (i + 1, 0, 0)),
            pl.BlockSpec((1, ROWS, LANES), lambda i: (i + 2, 0, 0)),
            pl.BlockSpec((K * K * K, LANES, NOUT), lambda i: (0, 0, 0)),
            pl.BlockSpec((1, NOUT), lambda i: (0, 0)),
        ],
        out_specs=pl.BlockSpec((1, OUT, 8, NOUT), lambda i: (i, 0, 0, 0)),
        out_shape=jax.ShapeDtypeStruct((OUT, OUT, 8, NOUT), jnp.float32),
    )(dense_p, dense_p, dense_p, w_bd, bias_p)


def _slice_body(src_ref, out_ref, sems):
    i = pl.program_id(0)
    copies = []
    for j in range(8):
        nz = 8 if j < 7 else OUT - 8 * j
        copies.append(pltpu.make_async_copy(
            src_ref.at[i, :, j, 0:nz, :],
            out_ref.at[i, :, pl.ds(8 * j, nz), :],
            sems.at[j]))
    for cp in copies:
        cp.start()
    for cp in copies:
        cp.wait()


@jax.jit
def _slice(out_p5):
    return pl.pallas_call(
        _slice_body,
        grid=(OUT,),
        in_specs=[pl.BlockSpec(memory_space=pl.ANY)],
        out_specs=pl.BlockSpec(memory_space=pl.ANY),
        out_shape=jax.ShapeDtypeStruct((OUT, OUT, OUT, C_OUT), jnp.float32),
        scratch_shapes=[pltpu.SemaphoreType.DMA((8,))],
    )(out_p5)


NPTS = 8192
PTS_PER_TILE = NPTS // 32
XS_PER_CHUNK = GRID // 4
CHUNK_ROWS = XS_PER_CHUNK * ROWS
DUMP = CHUNK_ROWS


@functools.partial(
    pl.kernel,
    out_type=jax.ShapeDtypeStruct((GRID, ROWS, LANES), jnp.float32),
    mesh=plsc.VectorSubcoreMesh(core_axis_name="c", subcore_axis_name="s",
                                num_cores=2, num_subcores=16),
    scratch_types=[
        pltpu.VMEM((40, LANES), jnp.float32),
        pltpu.VMEM((PTS_PER_TILE, LANES), jnp.float32),
        pltpu.VMEM((16, 128), jnp.int32),
        pltpu.VMEM((2, 128), jnp.int32),
        pltpu.VMEM_SHARED((CHUNK_ROWS + 16, LANES), jnp.float32),
    ],
)
def _scatter(pr_hbm, val_hbm, out_hbm, zero_v, vals_v, praw_v, psel_v, sp):
    c = lax.axis_index("c")
    s = lax.axis_index("s")
    pltpu.sync_copy(pr_hbm.at[pl.ds(s * 16, 16)], praw_v)
    zrow = jnp.zeros((16,), jnp.float32)
    dump = DUMP + lax.iota(jnp.int32, 16)

    def _zero_row(i, carry):
        for q in range(8):
            zero_v[i, 16 * q:16 * q + 16] = zrow
        return carry

    lax.fori_loop(0, 40, _zero_row, 0)
    for chunk_i in range(2):
        chunk = c * 2 + chunk_i
        base = chunk * CHUNK_ROWS
        for zb in range(ROWS // 40):
            pltpu.sync_copy(zero_v, sp.at[pl.ds(s * ROWS + zb * 40, 40)])
        plsc.subcore_barrier()
        for b in range(2):
            pltpu.sync_copy(
                val_hbm.at[pl.ds((s * 2 + b) * PTS_PER_TILE, PTS_PER_TILE)],
                vals_v)
            for h in range(2):
                for j in range(8):
                    g = praw_v[8 * b + h, 16 * j:16 * j + 16]
                    row = g - base
                    valid = (row >= 0) & (row < CHUNK_ROWS)
                    psel_v[h, 16 * j:16 * j + 16] = jnp.where(valid, row, dump)
            for h in range(2):
                pltpu.sync_copy(vals_v.at[pl.ds(128 * h, 128)],
                                sp.at[psel_v.at[h]], add=True)
        plsc.subcore_barrier()
        pltpu.sync_copy(sp.at[pl.ds(s * ROWS, ROWS)],
                        out_hbm.at[chunk * XS_PER_CHUNK + s])
        plsc.subcore_barrier()


def kernel(indices, values, kernel, bias):
    lin = indices[:, 1] * GRID + indices[:, 2]
    pr = indices[:, 0] * ROWS + lin // 8
    lb = lin % 8
    val128 = (jax.nn.one_hot(lb, 8, dtype=values.dtype)[:, :, None]
              * values[:, None, :]).reshape(NPTS, LANES)
    pr_pad = jnp.pad(pr.astype(jnp.int32).reshape(32, 256), ((0, 0), (0, 768)))
    dense_p = _scatter(pr_pad.reshape(256, 128), val128)
    w_flat = kernel.reshape(K * K * K, C_IN, C_OUT)
    eye8 = jnp.eye(8, dtype=w_flat.dtype)
    w_bd = jnp.einsum('ab,tio->taibo', eye8, w_flat).reshape(
        K * K * K, LANES, NOUT)
    bias_p = jnp.tile(bias, 8).reshape(1, NOUT)
    out_p = _conv(dense_p, w_bd, bias_p)
    return _slice(out_p.reshape(OUT, OUT, 8, 8, C_OUT))


# speedup vs baseline: 14.9911x; 14.9911x over previous
import functools

import jax
import jax.numpy as jnp
from jax import lax
from jax.experimental import pallas as pl
from jax.experimental.pallas import tpu as pltpu
from jax.experimental.pallas import tpu_sc as plsc

GRID = 64
C_IN = 16
C_OUT = 32
K = 3
OUT = GRID - K + 1
PLANE = GRID * GRID
ROWS = 520
M8 = OUT * GRID // 8
LANES = 8 * C_IN
NOUT = 8 * C_OUT


def _conv_body(d0_ref, d1_ref, d2_ref, w2_ref, w1_ref, b_ref, out_ref):
    slabs = (d0_ref, d1_ref, d2_ref)
    shifted = []
    for kx in range(K):
        s = slabs[kx][0]
        row = [s[0:ROWS - 1]]
        for kz in (1, 2):
            sh = jnp.concatenate(
                [s[0:ROWS - 1, 16 * kz:], s[1:ROWS, :16 * kz]], axis=1)
            row.append(sh)
        shifted.append(row)
    acc = jnp.zeros((M8, NOUT), dtype=jnp.float32)
    for kx in range(K):
        c01 = jnp.concatenate([shifted[kx][0], shifted[kx][1]], axis=1)
        for ky in range(K):
            t = kx * K + ky
            acc += jnp.dot(c01[8 * ky:8 * ky + M8, :], w2_ref[t],
                           preferred_element_type=jnp.float32)
            acc += jnp.dot(shifted[kx][2][8 * ky:8 * ky + M8, :], w1_ref[t],
                           preferred_element_type=jnp.float32)
    acc += b_ref[0]
    out_ref[0] = jnp.maximum(acc, 0.0).reshape(OUT, 8, NOUT)


@jax.jit
def _conv(dense_p, w2, w1, bias_p):
    return pl.pallas_call(
        _conv_body,
        grid=(OUT,),
        in_specs=[
            pl.BlockSpec((1, ROWS, LANES), lambda i: (i, 0, 0)),
            pl.BlockSpec((1, ROWS, LANES), lambda i: (i + 1, 0, 0)),
            pl.BlockSpec((1, ROWS, LANES), lambda i: (i + 2, 0, 0)),
            pl.BlockSpec((K * K, 2 * LANES, NOUT), lambda i: (0, 0, 0)),
            pl.BlockSpec((K * K, LANES, NOUT), lambda i: (0, 0, 0)),
            pl.BlockSpec((1, NOUT), lambda i: (0, 0)),
        ],
        out_specs=pl.BlockSpec((1, OUT, 8, NOUT), lambda i: (i, 0, 0, 0)),
        out_shape=jax.ShapeDtypeStruct((OUT, OUT, 8, NOUT), jnp.float32),
    )(dense_p, dense_p, dense_p, w2, w1, bias_p)


NPTS = 8192
PTS_PER_TILE = NPTS // 32
XS_PER_CHUNK = GRID // 4
CHUNK_ROWS = XS_PER_CHUNK * ROWS
DUMP = CHUNK_ROWS


@functools.partial(
    pl.kernel,
    out_type=jax.ShapeDtypeStruct((GRID, ROWS, LANES), jnp.float32),
    mesh=plsc.VectorSubcoreMesh(core_axis_name="c", subcore_axis_name="s",
                                num_cores=2, num_subcores=16),
    scratch_types=[
        pltpu.VMEM((40, LANES), jnp.float32),
        pltpu.VMEM((PTS_PER_TILE, LANES), jnp.float32),
        pltpu.VMEM((16, 128), jnp.int32),
        pltpu.VMEM((2, 128), jnp.int32),
        pltpu.VMEM_SHARED((CHUNK_ROWS + 16, LANES), jnp.float32),
    ],
)
def _scatter(pr_hbm, val_hbm, out_hbm, zero_v, vals_v, praw_v, psel_v, sp):
    c = lax.axis_index("c")
    s = lax.axis_index("s")
    pltpu.sync_copy(pr_hbm.at[pl.ds(s * 16, 16)], praw_v)
    zrow = jnp.zeros((16,), jnp.float32)
    dump = DUMP + lax.iota(jnp.int32, 16)

    def _zero_row(i, carry):
        for q in range(8):
            zero_v[i, 16 * q:16 * q + 16] = zrow
        return carry

    lax.fori_loop(0, 40, _zero_row, 0)
    for chunk_i in range(2):
        chunk = c * 2 + chunk_i
        base = chunk * CHUNK_ROWS
        for zb in range(ROWS // 40):
            pltpu.sync_copy(zero_v, sp.at[pl.ds(s * ROWS + zb * 40, 40)])
        plsc.subcore_barrier()
        for b in range(2):
            pltpu.sync_copy(
                val_hbm.at[pl.ds((s * 2 + b) * PTS_PER_TILE, PTS_PER_TILE)],
                vals_v)
            for h in range(2):
                for j in range(8):
                    g = praw_v[8 * b + h, 16 * j:16 * j + 16]
                    row = g - base
                    valid = (row >= 0) & (row < CHUNK_ROWS)
                    psel_v[h, 16 * j:16 * j + 16] = jnp.where(valid, row, dump)
            for h in range(2):
                pltpu.sync_copy(vals_v.at[pl.ds(128 * h, 128)],
                                sp.at[psel_v.at[h]], add=True)
        plsc.subcore_barrier()
        pltpu.sync_copy(sp.at[pl.ds(s * ROWS, ROWS)],
                        out_hbm.at[chunk * XS_PER_CHUNK + s])
        plsc.subcore_barrier()


def kernel(indices, values, kernel, bias):
    lin = indices[:, 1] * GRID + indices[:, 2]
    pr = indices[:, 0] * ROWS + lin // 8
    lb = lin % 8
    val128 = (jax.nn.one_hot(lb, 8, dtype=values.dtype)[:, :, None]
              * values[:, None, :]).reshape(NPTS, LANES)
    pr_pad = jnp.pad(pr.astype(jnp.int32).reshape(32, 256), ((0, 0), (0, 768)))
    dense_p = _scatter(pr_pad.reshape(256, 128), val128)
    w_flat = kernel.reshape(K * K * K, C_IN, C_OUT)
    eye8 = jnp.eye(8, dtype=w_flat.dtype)
    w_bd = jnp.einsum('ab,tio->taibo', eye8, w_flat).reshape(
        K, K, K, LANES, NOUT)
    w2 = jnp.concatenate([w_bd[:, :, 0], w_bd[:, :, 1]], axis=2).reshape(
        K * K, 2 * LANES, NOUT)
    w1 = w_bd[:, :, 2].reshape(K * K, LANES, NOUT)
    bias_p = jnp.tile(bias, 8).reshape(1, NOUT)
    out_p = _conv(dense_p, w2, w1, bias_p)
    return out_p.reshape(OUT, OUT, GRID, C_OUT)[:, :, :OUT, :]
